# baseline (speedup 1.0000x reference)
import jax
import jax.numpy as jnp
from jax.experimental import pallas as pl

N = 128
R = 7
O = 32


def _gcn_kernel(x_ref, wrel_ref, wroot_ref, bias_ref, out_ref):
    blk = x_ref[0]
    ones = jnp.ones((R, 1, N), dtype=jnp.bfloat16)
    deg = jax.lax.dot_general(
        ones, blk, (((2,), (1,)), ((0,), (0,))),
        preferred_element_type=jnp.float32)
    recip = (1.0 / jnp.maximum(deg, 1.0)).astype(jnp.bfloat16)
    m = jax.lax.dot_general(
        blk, wrel_ref[...], (((2,), (1,)), ((0,), (0,))),
        preferred_element_type=jnp.float32).astype(jnp.bfloat16)
    at = (blk * recip).reshape(R * N, N)
    out_rel = jax.lax.dot_general(
        at, m.reshape(R * N, O), (((0,), (0,)), ((), ())),
        preferred_element_type=jnp.float32)
    hsum = jnp.sum(blk, axis=0)
    root = jax.lax.dot_general(
        hsum, wroot_ref[...], (((1,), (0,)), ((), ())),
        preferred_element_type=jnp.float32)
    out_ref[0] = out_rel + root * (1.0 / R) + bias_ref[...]


@jax.jit
def kernel(x, W_rel, W_root, bias):
    B = x.shape[0]
    xt = jnp.transpose(x.astype(jnp.bfloat16), (0, 3, 1, 2))
    bias2 = bias.reshape(1, O)
    return pl.pallas_call(
        _gcn_kernel,
        grid=(B,),
        in_specs=[
            pl.BlockSpec((1, R, N, N), lambda b: (b, 0, 0, 0)),
            pl.BlockSpec((R, N, O), lambda b: (0, 0, 0)),
            pl.BlockSpec((N, O), lambda b: (0, 0)),
            pl.BlockSpec((1, O), lambda b: (0, 0)),
        ],
        out_specs=pl.BlockSpec((1, N, O), lambda b: (b, 0, 0)),
        out_shape=jax.ShapeDtypeStruct((B, N, O), jnp.float32),
    )(xt, W_rel.astype(jnp.bfloat16), W_root.astype(jnp.bfloat16), bias2)
